# baseline (speedup 1.0000x reference)
import jax
import jax.numpy as jnp
from jax.experimental import pallas as pl
from jax.experimental.pallas import tpu as pltpu

_SUB = 16
_NB = 4


def _hinge_body(y_ref, x_hbm, o_ref, buf, sem):
    b, c = x_hbm.shape
    nsteps = b // _SUB

    def chunk_copy(i, slot):
        return pltpu.make_async_copy(
            x_hbm.at[pl.ds(i * _SUB, _SUB), :], buf.at[slot], sem.at[slot]
        )

    for s in range(_NB):
        chunk_copy(s, s).start()

    def gstep(g, carry):
        for slot in range(_NB):
            i = g * _NB + slot
            chunk_copy(i, slot).wait()
            x = buf[slot]
            yv = y_ref[pl.ds(i * _SUB, _SUB), :]
            cols = jax.lax.broadcasted_iota(jnp.int32, x.shape, 1)
            oy = jnp.sum(jnp.where(cols == yv, x, 0.0), axis=1, keepdims=True)
            s_ = jnp.sum(jnp.maximum(x - (oy - 1.0), 0.0), axis=1,
                         keepdims=True)
            o_ref[pl.ds(i * _SUB, _SUB), :] = (s_ - 1.0) / c

            @pl.when(i + _NB < nsteps)
            def _():
                chunk_copy(i + _NB, slot).start()
        return carry

    jax.lax.fori_loop(0, nsteps // _NB, gstep, 0)


def kernel(output, y):
    b, c = output.shape
    y2 = y.astype(jnp.int32).reshape(b, 1)
    out = pl.pallas_call(
        _hinge_body,
        in_specs=[
            pl.BlockSpec(memory_space=pltpu.VMEM),
            pl.BlockSpec(memory_space=pl.ANY),
        ],
        out_specs=pl.BlockSpec(memory_space=pltpu.VMEM),
        out_shape=jax.ShapeDtypeStruct((b, 1), jnp.float32),
        scratch_shapes=[
            pltpu.VMEM((_NB, _SUB, c), jnp.float32),
            pltpu.SemaphoreType.DMA((_NB,)),
        ],
        compiler_params=pltpu.CompilerParams(
            vmem_limit_bytes=100 * 1024 * 1024,
        ),
    )(y2, output)
    return out.reshape(b)
